# baseline (speedup 1.0000x reference)
import functools

import jax
import jax.numpy as jnp
from jax import lax
from jax.experimental import pallas as pl
from jax.experimental.pallas import tpu as pltpu
from jax.experimental.pallas import tpu_sc as plsc

N = 10000
E = 320000
IN_CH = 128
H = 8

NC = 2
NS = 16
NW = NC * NS
EPT = E // NW
K = 2000
NCHUNK = EPT // K
GPC = K // 16
NPAD = 10240
RPS = NPAD // NS

BN = 1000


def _prep_body(x_ref, w_ref, as_ref, ad_ref, xp_ref, mx_ref):
    i = pl.program_id(0)
    xp = jnp.dot(x_ref[...], w_ref[...], preferred_element_type=jnp.float32)
    xp_ref[...] = xp
    s = jnp.max(xp * as_ref[...], axis=0)
    d = jnp.max(xp * ad_ref[...], axis=0)
    sd = jnp.stack([s, d])

    @pl.when(i == 0)
    def _():
        mx_ref[...] = sd

    @pl.when(i > 0)
    def _():
        mx_ref[...] = jnp.maximum(mx_ref[...], sd)


def _prep(x, w, asv, adv):
    return pl.pallas_call(
        _prep_body,
        grid=(N // BN,),
        in_specs=[
            pl.BlockSpec((BN, IN_CH), lambda i: (i, 0)),
            pl.BlockSpec((IN_CH, H), lambda i: (0, 0)),
            pl.BlockSpec((1, H), lambda i: (0, 0)),
            pl.BlockSpec((1, H), lambda i: (0, 0)),
        ],
        out_specs=[
            pl.BlockSpec((BN, H), lambda i: (i, 0)),
            pl.BlockSpec((2, H), lambda i: (0, 0)),
        ],
        out_shape=[
            jax.ShapeDtypeStruct((N, H), jnp.float32),
            jax.ShapeDtypeStruct((2, H), jnp.float32),
        ],
    )(x, w, asv, adv)


def _edge_kernel_body(xp_hbm, src_hbm, dst_hbm, att_hbm, out_hbm,
                      xp_v, src_v, dst_v, val0, att_v, acc_sh):
    c = lax.axis_index("c")
    s = lax.axis_index("s")
    w = c * NS + s

    pltpu.sync_copy(xp_hbm, xp_v)
    pltpu.sync_copy(att_hbm, att_v)

    zeros16 = jnp.zeros((16,), jnp.float32)

    def _zero_row(i, _):
        val0[i, :] = zeros16
        return 0

    lax.fori_loop(0, RPS, _zero_row, 0)
    pltpu.sync_copy(val0.at[pl.ds(0, RPS)], acc_sh.at[pl.ds(s * RPS, RPS)])
    plsc.subcore_barrier()

    aS = [att_v[h] for h in range(H)]
    aD = [att_v[H + h] for h in range(H)]
    Bv = [att_v[2 * H + h] for h in range(H)]
    iota = lax.iota(jnp.int32, 16)
    cols = [iota * 0 + j for j in range(16)]
    for ci in range(NCHUNK):
        base = w * EPT + ci * K
        pltpu.sync_copy(src_hbm.at[pl.ds(base, K)], src_v)
        pltpu.sync_copy(dst_hbm.at[pl.ds(base, K)], dst_v)

        @plsc.parallel_loop(0, GPC, step=1, unroll=1)
        def _group(g):
            s16 = src_v[pl.ds(g * 16, 16)]
            d16 = dst_v[pl.ds(g * 16, 16)]
            is0 = s16 * H
            id0 = d16 * H
            row = g * 16 + iota
            for h in range(H):
                xs = plsc.load_gather(xp_v, [is0 + h])
                xd = plsc.load_gather(xp_v, [id0 + h])
                pre = xs * aS[h] + xd * aD[h]
                alpha = jnp.maximum(pre, 0.2 * pre)
                ex = jnp.exp(alpha - Bv[h])
                plsc.store_scatter(val0, [row, cols[h]], ex)
                plsc.store_scatter(val0, [row, cols[H + h]], ex * xs)

        pltpu.sync_copy(val0, acc_sh.at[dst_v], add=True)

    plsc.subcore_barrier()
    pltpu.sync_copy(acc_sh.at[pl.ds(s * RPS, RPS)],
                    out_hbm.at[c, pl.ds(s * RPS, RPS)])


def _edges(xp_flat, src, dst, att):
    mesh = plsc.VectorSubcoreMesh(core_axis_name="c", subcore_axis_name="s",
                                  num_cores=NC, num_subcores=NS)

    f = pl.kernel(
        _edge_kernel_body,
        out_type=jax.ShapeDtypeStruct((NC, NPAD, 16), jnp.float32),
        mesh=mesh,
        scratch_types=[
            pltpu.VMEM((N * H,), jnp.float32),
            pltpu.VMEM((K,), jnp.int32),
            pltpu.VMEM((K,), jnp.int32),
            pltpu.VMEM((K, 16), jnp.float32),
            pltpu.VMEM((3 * H, 16), jnp.float32),
            pltpu.VMEM_SHARED((NPAD, 16), jnp.float32),
        ],
        compiler_params=pltpu.CompilerParams(needs_layout_passes=False,
                                             use_tc_tiling_on_sc=False),
    )
    return f(xp_flat, src, dst, att)


def _final_body(acc_ref, gb_ref, lw_ref, lb_ref, bp_ref, out_ref):
    a = acc_ref[0] + acc_ref[1]
    denom = a[:, 0:H]
    numer = a[:, H:2 * H]
    hmid = numer / (denom + 1e-16) + gb_ref[...]
    t = jnp.sum(hmid * lw_ref[...], axis=1, keepdims=True) + lb_ref[...]
    out_ref[...] = jnp.maximum(t, 0.0) + bp_ref[...]


def _final(acc, gb, lw, lb, bp):
    return pl.pallas_call(
        _final_body,
        grid=(N // BN,),
        in_specs=[
            pl.BlockSpec((NC, BN, 16), lambda i: (0, i, 0)),
            pl.BlockSpec((1, H), lambda i: (0, 0)),
            pl.BlockSpec((1, H), lambda i: (0, 0)),
            pl.BlockSpec((1, 1), lambda i: (0, 0)),
            pl.BlockSpec((1, 1), lambda i: (0, 0)),
        ],
        out_specs=pl.BlockSpec((BN, 1), lambda i: (i, 0)),
        out_shape=jax.ShapeDtypeStruct((N, 1), jnp.float32),
    )(acc, gb, lw, lb, bp)


def kernel(x, edge_index, W, att_src, att_dst, gat_bias, lin_w, lin_b, bias_param):
    src = edge_index[0].astype(jnp.int32)
    dst = edge_index[1].astype(jnp.int32)
    asv = att_src.reshape(1, H)
    adv = att_dst.reshape(1, H)

    xp, sdmax = _prep(x, W, asv, adv)

    t = sdmax[0] + sdmax[1]
    B = jnp.maximum(t, 0.2 * t)
    att = jnp.concatenate([
        jnp.broadcast_to(asv.reshape(H, 1), (H, 16)),
        jnp.broadcast_to(adv.reshape(H, 1), (H, 16)),
        jnp.broadcast_to(B.reshape(H, 1), (H, 16)),
    ], axis=0)

    acc = _edges(xp.reshape(-1), src, dst, att)

    return _final(acc, gat_bias.reshape(1, H), lin_w,
                  lin_b.reshape(1, 1), bias_param.reshape(1, 1))


# speedup vs baseline: 1.0889x; 1.0889x over previous
import functools

import jax
import jax.numpy as jnp
from jax import lax
from jax.experimental import pallas as pl
from jax.experimental.pallas import tpu as pltpu
from jax.experimental.pallas import tpu_sc as plsc

N = 10000
E = 320000
IN_CH = 128
H = 8

NC = 2
NS = 16
NW = NC * NS
EPT = E // NW
K = 2000
NCHUNK = EPT // K
GPC = K // 16
NPAD = 10240
RPS = NPAD // NS

BN = 1000


def _prep_body(x_ref, w_ref, as_ref, ad_ref, xp_ref, mx_ref, att_ref):
    i = pl.program_id(0)
    xp = jnp.dot(x_ref[...], w_ref[...], preferred_element_type=jnp.float32)
    xp_ref[...] = xp
    s = jnp.max(xp * as_ref[...], axis=0)
    d = jnp.max(xp * ad_ref[...], axis=0)
    sd = jnp.stack([s, d])

    @pl.when(i == 0)
    def _():
        mx_ref[...] = sd

    @pl.when(i > 0)
    def _():
        mx_ref[...] = jnp.maximum(mx_ref[...], sd)

    @pl.when(i == N // BN - 1)
    def _():
        mx = mx_ref[...]
        t = mx[0:1, :] + mx[1:2, :]
        b = jnp.maximum(t, 0.2 * t)
        pad = jnp.zeros((1, H), jnp.float32)
        rows = jnp.concatenate([
            jnp.concatenate([as_ref[...], pad], axis=1),
            jnp.concatenate([ad_ref[...], pad], axis=1),
            jnp.concatenate([b, pad], axis=1),
        ], axis=0)
        att_ref[...] = rows


def _prep(x, w, asv, adv):
    return pl.pallas_call(
        _prep_body,
        grid=(N // BN,),
        in_specs=[
            pl.BlockSpec((BN, IN_CH), lambda i: (i, 0)),
            pl.BlockSpec((IN_CH, H), lambda i: (0, 0)),
            pl.BlockSpec((1, H), lambda i: (0, 0)),
            pl.BlockSpec((1, H), lambda i: (0, 0)),
        ],
        out_specs=[
            pl.BlockSpec((BN, H), lambda i: (i, 0)),
            pl.BlockSpec((2, H), lambda i: (0, 0)),
            pl.BlockSpec((3, 2 * H), lambda i: (0, 0)),
        ],
        out_shape=[
            jax.ShapeDtypeStruct((N, H), jnp.float32),
            jax.ShapeDtypeStruct((2, H), jnp.float32),
            jax.ShapeDtypeStruct((3, 2 * H), jnp.float32),
        ],
    )(x, w, asv, adv)


def _edge_kernel_body(xp_hbm, ei_hbm, att_hbm, out_hbm,
                      xp_v, src_v, dst_v, val0, att_v, acc_sh):
    c = lax.axis_index("c")
    s = lax.axis_index("s")
    w = c * NS + s

    pltpu.sync_copy(xp_hbm, xp_v)
    pltpu.sync_copy(att_hbm, att_v)

    zeros16 = jnp.zeros((16,), jnp.float32)

    def _zero_row(i, _):
        val0[i, :] = zeros16
        return 0

    lax.fori_loop(0, RPS, _zero_row, 0)
    pltpu.sync_copy(val0.at[pl.ds(0, RPS)], acc_sh.at[pl.ds(s * RPS, RPS)])
    plsc.subcore_barrier()

    iota = lax.iota(jnp.int32, 16)
    cols = [iota * 0 + j for j in range(16)]
    aS = [plsc.load_gather(att_v, [cols[0], cols[h]]) for h in range(H)]
    aD = [plsc.load_gather(att_v, [cols[1], cols[h]]) for h in range(H)]
    Bv = [plsc.load_gather(att_v, [cols[2], cols[h]]) for h in range(H)]
    for ci in range(NCHUNK):
        base = w * EPT + ci * K
        pltpu.sync_copy(ei_hbm.at[0, pl.ds(base, K)], src_v)
        pltpu.sync_copy(ei_hbm.at[1, pl.ds(base, K)], dst_v)

        @plsc.parallel_loop(0, GPC, step=1, unroll=1)
        def _group(g):
            s16 = src_v[pl.ds(g * 16, 16)]
            d16 = dst_v[pl.ds(g * 16, 16)]
            is0 = s16 * H
            id0 = d16 * H
            row = g * 16 + iota
            for h in range(H):
                xs = plsc.load_gather(xp_v, [is0 + h])
                xd = plsc.load_gather(xp_v, [id0 + h])
                pre = xs * aS[h] + xd * aD[h]
                alpha = jnp.maximum(pre, 0.2 * pre)
                ex = jnp.exp(alpha - Bv[h])
                plsc.store_scatter(val0, [row, cols[h]], ex)
                plsc.store_scatter(val0, [row, cols[H + h]], ex * xs)

        pltpu.sync_copy(val0, acc_sh.at[dst_v], add=True)

    plsc.subcore_barrier()
    pltpu.sync_copy(acc_sh.at[pl.ds(s * RPS, RPS)],
                    out_hbm.at[c, pl.ds(s * RPS, RPS)])


def _edges(xp_flat, ei, att):
    mesh = plsc.VectorSubcoreMesh(core_axis_name="c", subcore_axis_name="s",
                                  num_cores=NC, num_subcores=NS)

    f = pl.kernel(
        _edge_kernel_body,
        out_type=jax.ShapeDtypeStruct((NC, NPAD, 16), jnp.float32),
        mesh=mesh,
        scratch_types=[
            pltpu.VMEM((N * H,), jnp.float32),
            pltpu.VMEM((K,), jnp.int32),
            pltpu.VMEM((K,), jnp.int32),
            pltpu.VMEM((K, 16), jnp.float32),
            pltpu.VMEM((3, 2 * H), jnp.float32),
            pltpu.VMEM_SHARED((NPAD, 16), jnp.float32),
        ],
        compiler_params=pltpu.CompilerParams(needs_layout_passes=False,
                                             use_tc_tiling_on_sc=False),
    )
    return f(xp_flat, ei, att)


def _final_body(acc_ref, gb_ref, lw_ref, lb_ref, bp_ref, out_ref):
    a = acc_ref[0] + acc_ref[1]
    denom = a[:, 0:H]
    numer = a[:, H:2 * H]
    hmid = numer / (denom + 1e-16) + gb_ref[...]
    t = jnp.sum(hmid * lw_ref[...], axis=1, keepdims=True) + lb_ref[...]
    out_ref[...] = jnp.maximum(t, 0.0) + bp_ref[...]


def _final(acc, gb, lw, lb, bp):
    return pl.pallas_call(
        _final_body,
        grid=(N // BN,),
        in_specs=[
            pl.BlockSpec((NC, BN, 16), lambda i: (0, i, 0)),
            pl.BlockSpec((1, H), lambda i: (0, 0)),
            pl.BlockSpec((1, H), lambda i: (0, 0)),
            pl.BlockSpec((1, 1), lambda i: (0, 0)),
            pl.BlockSpec((1, 1), lambda i: (0, 0)),
        ],
        out_specs=pl.BlockSpec((BN, 1), lambda i: (i, 0)),
        out_shape=jax.ShapeDtypeStruct((N, 1), jnp.float32),
    )(acc, gb, lw, lb, bp)


def kernel(x, edge_index, W, att_src, att_dst, gat_bias, lin_w, lin_b, bias_param):
    asv = att_src.reshape(1, H)
    adv = att_dst.reshape(1, H)

    xp, _, att = _prep(x, W, asv, adv)

    acc = _edges(xp.reshape(-1), edge_index.astype(jnp.int32), att)

    return _final(acc, gat_bias.reshape(1, H), lin_w,
                  lin_b.reshape(1, 1), bias_param.reshape(1, 1))


# speedup vs baseline: 1.0914x; 1.0023x over previous
import functools

import jax
import jax.numpy as jnp
from jax import lax
from jax.experimental import pallas as pl
from jax.experimental.pallas import tpu as pltpu
from jax.experimental.pallas import tpu_sc as plsc

N = 10000
E = 320000
IN_CH = 128
H = 8

NC = 2
NS = 16
NW = NC * NS
EPT = E // NW
K = 2000
NCHUNK = EPT // K
GPC = K // 16
NPAD = 10240
RPS = NPAD // NS

BN = 1000


def _prep_body(x_ref, w_ref, as_ref, ad_ref, xp_ref, mx_ref, att_ref):
    i = pl.program_id(0)
    xp = jnp.dot(x_ref[...], w_ref[...], preferred_element_type=jnp.float32)
    xp_ref[...] = xp
    s = jnp.max(xp * as_ref[...], axis=0)
    d = jnp.max(xp * ad_ref[...], axis=0)
    sd = jnp.stack([s, d])

    @pl.when(i == 0)
    def _():
        mx_ref[...] = sd

    @pl.when(i > 0)
    def _():
        mx_ref[...] = jnp.maximum(mx_ref[...], sd)

    @pl.when(i == N // BN - 1)
    def _():
        mx = mx_ref[...]
        t = mx[0:1, :] + mx[1:2, :]
        b = jnp.maximum(t, 0.2 * t)
        pad = jnp.zeros((1, H), jnp.float32)
        rows = jnp.concatenate([
            jnp.concatenate([as_ref[...], pad], axis=1),
            jnp.concatenate([ad_ref[...], pad], axis=1),
            jnp.concatenate([b, pad], axis=1),
        ], axis=0)
        att_ref[...] = rows


def _prep(x, w, asv, adv):
    return pl.pallas_call(
        _prep_body,
        grid=(N // BN,),
        in_specs=[
            pl.BlockSpec((BN, IN_CH), lambda i: (i, 0)),
            pl.BlockSpec((IN_CH, H), lambda i: (0, 0)),
            pl.BlockSpec((1, H), lambda i: (0, 0)),
            pl.BlockSpec((1, H), lambda i: (0, 0)),
        ],
        out_specs=[
            pl.BlockSpec((BN, H), lambda i: (i, 0)),
            pl.BlockSpec((2, H), lambda i: (0, 0)),
            pl.BlockSpec((3, 2 * H), lambda i: (0, 0)),
        ],
        out_shape=[
            jax.ShapeDtypeStruct((N, H), jnp.float32),
            jax.ShapeDtypeStruct((2, H), jnp.float32),
            jax.ShapeDtypeStruct((3, 2 * H), jnp.float32),
        ],
    )(x, w, asv, adv)


def _edge_kernel_body(xp_hbm, ei_hbm, att_hbm, out_hbm,
                      xp_v, src_v, dst_v, val0, att_v, acc_sh):
    c = lax.axis_index("c")
    s = lax.axis_index("s")
    w = c * NS + s

    pltpu.sync_copy(xp_hbm, xp_v)
    pltpu.sync_copy(att_hbm, att_v)

    zeros16 = jnp.zeros((16,), jnp.float32)

    def _zero_row(i, _):
        val0[i, :] = zeros16
        return 0

    lax.fori_loop(0, RPS, _zero_row, 0)
    pltpu.sync_copy(val0.at[pl.ds(0, RPS)], acc_sh.at[pl.ds(s * RPS, RPS)])
    plsc.subcore_barrier()

    iota = lax.iota(jnp.int32, 16)
    cols = [iota * 0 + j for j in range(16)]
    aS = [plsc.load_gather(att_v, [cols[h]]) for h in range(H)]
    aD = [plsc.load_gather(att_v, [cols[h] + 2 * H]) for h in range(H)]
    Bv = [plsc.load_gather(att_v, [cols[h] + 4 * H]) for h in range(H)]
    for ci in range(NCHUNK):
        base = w * EPT + ci * K
        pltpu.sync_copy(ei_hbm.at[pl.ds(base, K)], src_v)
        pltpu.sync_copy(ei_hbm.at[pl.ds(E + base, K)], dst_v)

        @plsc.parallel_loop(0, GPC, step=1, unroll=1)
        def _group(g):
            s16 = src_v[pl.ds(g * 16, 16)]
            d16 = dst_v[pl.ds(g * 16, 16)]
            is0 = s16 * H
            id0 = d16 * H
            row = g * 16 + iota
            for h in range(H):
                xs = plsc.load_gather(xp_v, [is0 + h])
                xd = plsc.load_gather(xp_v, [id0 + h])
                pre = xs * aS[h] + xd * aD[h]
                alpha = jnp.maximum(pre, 0.2 * pre)
                ex = jnp.exp(alpha - Bv[h])
                plsc.store_scatter(val0, [row, cols[h]], ex)
                plsc.store_scatter(val0, [row, cols[H + h]], ex * xs)

        pltpu.sync_copy(val0, acc_sh.at[dst_v], add=True)

    plsc.subcore_barrier()
    pltpu.sync_copy(acc_sh.at[pl.ds(s * RPS, RPS)],
                    out_hbm.at[c, pl.ds(s * RPS, RPS)])


def _edges(xp_flat, ei_flat, att_flat):
    mesh = plsc.VectorSubcoreMesh(core_axis_name="c", subcore_axis_name="s",
                                  num_cores=NC, num_subcores=NS)

    f = pl.kernel(
        _edge_kernel_body,
        out_type=jax.ShapeDtypeStruct((NC, NPAD, 16), jnp.float32),
        mesh=mesh,
        scratch_types=[
            pltpu.VMEM((N * H,), jnp.float32),
            pltpu.VMEM((K,), jnp.int32),
            pltpu.VMEM((K,), jnp.int32),
            pltpu.VMEM((K, 16), jnp.float32),
            pltpu.VMEM((3 * 2 * H,), jnp.float32),
            pltpu.VMEM_SHARED((NPAD, 16), jnp.float32),
        ],
        compiler_params=pltpu.CompilerParams(needs_layout_passes=False,
                                             use_tc_tiling_on_sc=False),
    )
    return f(xp_flat, ei_flat, att_flat)


def _final_body(acc_ref, gb_ref, lw_ref, lb_ref, bp_ref, out_ref):
    a = acc_ref[0] + acc_ref[1]
    denom = a[:, 0:H]
    numer = a[:, H:2 * H]
    hmid = numer / (denom + 1e-16) + gb_ref[...]
    t = jnp.sum(hmid * lw_ref[...], axis=1, keepdims=True) + lb_ref[...]
    out_ref[...] = jnp.maximum(t, 0.0) + bp_ref[...]


def _final(acc, gb, lw, lb, bp):
    return pl.pallas_call(
        _final_body,
        grid=(N // BN,),
        in_specs=[
            pl.BlockSpec((NC, BN, 16), lambda i: (0, i, 0)),
            pl.BlockSpec((1, H), lambda i: (0, 0)),
            pl.BlockSpec((1, H), lambda i: (0, 0)),
            pl.BlockSpec((1, 1), lambda i: (0, 0)),
            pl.BlockSpec((1, 1), lambda i: (0, 0)),
        ],
        out_specs=pl.BlockSpec((BN, 1), lambda i: (i, 0)),
        out_shape=jax.ShapeDtypeStruct((N, 1), jnp.float32),
    )(acc, gb, lw, lb, bp)


def kernel(x, edge_index, W, att_src, att_dst, gat_bias, lin_w, lin_b, bias_param):
    asv = att_src.reshape(1, H)
    adv = att_dst.reshape(1, H)

    xp, _, att = _prep(x, W, asv, adv)

    ei = edge_index.astype(jnp.int32)
    acc = _edges(xp.reshape(-1), ei.reshape(-1), att.reshape(-1))

    return _final(acc, gat_bias.reshape(1, H), lin_w,
                  lin_b.reshape(1, 1), bias_param.reshape(1, 1))
